# baseline (speedup 1.0000x reference)
import functools

import jax
import jax.numpy as jnp
from jax import lax
from jax.experimental import pallas as pl
from jax.experimental.pallas import tpu as pltpu
from jax.experimental.pallas import tpu_sc as plsc

B = 16
S = 4096
LANES = 16
CHUNKS = S // LANES
GROUPS = CHUNKS // LANES


def _make_sc_kernel():
  mesh = plsc.VectorSubcoreMesh(
      core_axis_name="c", subcore_axis_name="s", num_cores=1)

  @functools.partial(
      pl.kernel,
      mesh=mesh,
      out_type=jax.ShapeDtypeStruct((B, S), jnp.float32),
      scratch_types=[
          pltpu.VMEM((S,), jnp.float32),
          pltpu.VMEM((CHUNKS,), jnp.float32),
          pltpu.VMEM((CHUNKS,), jnp.float32),
          pltpu.SemaphoreType.DMA,
          pltpu.SemaphoreType.DMA,
      ],
      compiler_params=pltpu.CompilerParams(needs_layout_passes=False),
  )
  def cumsum_kernel(pad_hbm, out_hbm, buf, totals, offs, sem_in, sem_out):
    row = lax.axis_index("s")

    pltpu.async_copy(pad_hbm.at[row], buf, sem_in).wait()

    def scan_chunk(i, _):
      base = i * LANES
      buf[pl.ds(base, LANES)] = plsc.cumsum(buf[pl.ds(base, LANES)])
      return 0

    lax.fori_loop(0, CHUNKS, scan_chunk, 0, unroll=8)

    stride16 = lax.iota(jnp.int32, LANES) * LANES + (LANES - 1)

    def gather_totals(g, _):
      totals[pl.ds(g * LANES, LANES)] = plsc.load_gather(
          buf, [stride16 + g * (LANES * LANES)])
      return 0

    lax.fori_loop(0, GROUPS, gather_totals, 0, unroll=4)

    lane15 = jnp.full((LANES,), LANES - 1, jnp.int32)

    def scan_totals(g, carry):
      base = g * LANES
      v = totals[pl.ds(base, LANES)]
      incl = plsc.cumsum(v) + carry
      totals[pl.ds(base, LANES)] = incl
      offs[pl.ds(base, LANES)] = incl - v
      return plsc.load_gather(totals, [lane15 + base])

    lax.fori_loop(0, GROUPS, scan_totals,
                  jnp.full((LANES,), -1.0, jnp.float32))

    def add_offsets(i, _):
      base = i * LANES
      o = plsc.load_gather(offs, [jnp.full((LANES,), 0, jnp.int32) + i])
      buf[pl.ds(base, LANES)] = buf[pl.ds(base, LANES)] + o
      return 0

    lax.fori_loop(0, CHUNKS, add_offsets, 0, unroll=8)

    pltpu.async_copy(buf, out_hbm.at[row], sem_out).wait()

  return cumsum_kernel


_sc_cumsum = _make_sc_kernel()


@jax.jit
def kernel(pad_masks):
  return _sc_cumsum(pad_masks)


# speedup vs baseline: 1.0770x; 1.0770x over previous
import functools

import jax
import jax.numpy as jnp
from jax import lax
from jax.experimental import pallas as pl
from jax.experimental.pallas import tpu as pltpu
from jax.experimental.pallas import tpu_sc as plsc

B = 16
S = 4096
LANES = 16
CHUNKS = S // LANES


def _make_sc_kernel():
  mesh = plsc.VectorSubcoreMesh(
      core_axis_name="c", subcore_axis_name="s", num_cores=1)

  @functools.partial(
      pl.kernel,
      mesh=mesh,
      out_type=jax.ShapeDtypeStruct((B, S), jnp.float32),
      scratch_types=[
          pltpu.VMEM((S,), jnp.float32),
          pltpu.SemaphoreType.DMA,
      ],
      compiler_params=pltpu.CompilerParams(needs_layout_passes=False),
  )
  def cumsum_kernel(pad_hbm, out_hbm, buf, sem):
    row = lax.axis_index("s")

    pltpu.async_copy(pad_hbm.at[row], buf, sem).wait()

    def scan_body(i, carry):
      base = i * LANES
      v = buf[pl.ds(base, LANES)]
      buf[pl.ds(base, LANES)] = plsc.cumsum(v) + carry
      return carry + jnp.sum(v)

    lax.fori_loop(0, CHUNKS, scan_body, jnp.float32(-1.0), unroll=4)

    pltpu.sync_copy(buf, out_hbm.at[row])

  return cumsum_kernel


_sc_cumsum = _make_sc_kernel()


@jax.jit
def kernel(pad_masks):
  return _sc_cumsum(pad_masks)


# speedup vs baseline: 1.1008x; 1.0221x over previous
import functools

import jax
import jax.numpy as jnp
from jax import lax
from jax.experimental import pallas as pl
from jax.experimental.pallas import tpu as pltpu
from jax.experimental.pallas import tpu_sc as plsc

B = 16
S = 4096
LANES = 16
CHUNKS = S // LANES


def _make_sc_kernel():
  mesh = plsc.VectorSubcoreMesh(
      core_axis_name="c", subcore_axis_name="s", num_cores=1)

  @functools.partial(
      pl.kernel,
      mesh=mesh,
      out_type=jax.ShapeDtypeStruct((B, S), jnp.float32),
      scratch_types=[
          pltpu.VMEM((S,), jnp.float32),
          pltpu.SemaphoreType.DMA,
      ],
      compiler_params=pltpu.CompilerParams(needs_layout_passes=False),
  )
  def cumsum_kernel(pad_hbm, out_hbm, buf, sem):
    row = lax.axis_index("s")

    pltpu.async_copy(pad_hbm.at[row], buf, sem).wait()

    def scan_body(i, carry):
      base = i * LANES
      v = buf[pl.ds(base, LANES)]
      buf[pl.ds(base, LANES)] = plsc.cumsum(v) + carry
      return carry + jnp.sum(v)

    lax.fori_loop(0, CHUNKS, scan_body, jnp.float32(-1.0), unroll=8)

    pltpu.sync_copy(buf, out_hbm.at[row])

  return cumsum_kernel


_sc_cumsum = _make_sc_kernel()


@jax.jit
def kernel(pad_masks):
  return _sc_cumsum(pad_masks)


# speedup vs baseline: 1.1093x; 1.0078x over previous
import functools

import jax
import jax.numpy as jnp
from jax import lax
from jax.experimental import pallas as pl
from jax.experimental.pallas import tpu as pltpu
from jax.experimental.pallas import tpu_sc as plsc

B = 16
S = 4096
LANES = 16
CHUNKS = S // LANES


def _make_sc_kernel():
  mesh = plsc.VectorSubcoreMesh(
      core_axis_name="c", subcore_axis_name="s", num_cores=1)

  @functools.partial(
      pl.kernel,
      mesh=mesh,
      out_type=jax.ShapeDtypeStruct((B, S), jnp.float32),
      scratch_types=[
          pltpu.VMEM((S,), jnp.float32),
          pltpu.SemaphoreType.DMA,
      ],
      compiler_params=pltpu.CompilerParams(needs_layout_passes=False),
  )
  def cumsum_kernel(pad_hbm, out_hbm, buf, sem):
    row = lax.axis_index("s")

    pltpu.async_copy(pad_hbm.at[row], buf, sem).wait()

    def scan_body(i, carry):
      base = i * LANES
      v = buf[pl.ds(base, LANES)]
      buf[pl.ds(base, LANES)] = plsc.cumsum(v) + carry
      return carry + jnp.sum(v)

    lax.fori_loop(0, CHUNKS, scan_body, jnp.float32(-1.0), unroll=16)

    pltpu.sync_copy(buf, out_hbm.at[row])

  return cumsum_kernel


_sc_cumsum = _make_sc_kernel()


@jax.jit
def kernel(pad_masks):
  return _sc_cumsum(pad_masks)
